# baseline (speedup 1.0000x reference)
import jax
import jax.numpy as jnp
from jax.experimental import pallas as pl
from jax.experimental.pallas import tpu as pltpu
from jax.sharding import Mesh, PartitionSpec as P

N_EMB = 1024
EMB_DIM = 64
COMMITMENT_COST = 0.25


def _vq_body(x_ref, cb_ref, oh_ref, q_ref, loss_ref, counts_ref):
    b = pl.program_id(0)
    xT = x_ref[0]
    cb = cb_ref[...]

    sx = jnp.sum(xT * xT, axis=0)
    scb = jnp.sum(cb * cb, axis=1)
    m = jax.lax.dot_general(cb, xT, (((1,), (0,)), ((), ())),
                            preferred_element_type=jnp.float32)
    dist = (sx[None, :] + scb[:, None]) - 2.0 * m
    idx = jnp.argmin(dist, axis=0)

    eiota = jax.lax.broadcasted_iota(jnp.int32, dist.shape, 0)
    ohT = (eiota == idx[None, :]).astype(jnp.float32)
    oh_ref[0] = ohT

    qT = jax.lax.dot_general(cb, ohT, (((0,), (0,)), ((), ())),
                             preferred_element_type=jnp.float32)
    q_ref[0] = qT

    part_loss = jnp.sum((qT - xT) ** 2).reshape(1, 1)
    ones_n = jnp.ones((ohT.shape[1], 8), jnp.float32)
    part_counts = jax.lax.dot_general(ohT, ones_n, (((1,), (0,)), ((), ())),
                                      preferred_element_type=jnp.float32)

    @pl.when(b == 0)
    def _init():
        loss_ref[...] = part_loss
        counts_ref[...] = part_counts

    @pl.when(b > 0)
    def _acc():
        loss_ref[...] += part_loss
        counts_ref[...] += part_counts


def _vq_shard(x3, cb):
    bs, C, HW = x3.shape
    oh, q3, lsum, counts = pl.pallas_call(
        _vq_body,
        grid=(bs,),
        in_specs=[
            pl.BlockSpec((1, C, HW), lambda b: (b, 0, 0)),
            pl.BlockSpec((N_EMB, EMB_DIM), lambda b: (0, 0)),
        ],
        out_specs=[
            pl.BlockSpec((1, N_EMB, HW), lambda b: (b, 0, 0)),
            pl.BlockSpec((1, C, HW), lambda b: (b, 0, 0)),
            pl.BlockSpec((1, 1), lambda b: (0, 0)),
            pl.BlockSpec((N_EMB, 8), lambda b: (0, 0)),
        ],
        out_shape=[
            jax.ShapeDtypeStruct((bs, N_EMB, HW), jnp.float32),
            jax.ShapeDtypeStruct((bs, C, HW), jnp.float32),
            jax.ShapeDtypeStruct((1, 1), jnp.float32),
            jax.ShapeDtypeStruct((N_EMB, 8), jnp.float32),
        ],
        compiler_params=pltpu.CompilerParams(
            dimension_semantics=("arbitrary",),
        ),
    )(x3, cb)
    lsum = jax.lax.psum(lsum, "b")
    counts = jax.lax.psum(counts, "b")
    return oh, q3, lsum, counts


def kernel(inputs, codebook):
    B, C, H, W = inputs.shape
    HW = H * W
    x3 = inputs.reshape(B, C, HW)

    devs = jax.devices()
    nd = len(devs)
    while B % nd:
        nd -= 1
    mesh = Mesh(devs[:nd], ("b",))
    oh, q3, lsum, counts = jax.shard_map(
        _vq_shard,
        mesh=mesh,
        in_specs=(P("b", None, None), P(None, None)),
        out_specs=(P("b", None, None), P("b", None, None), P(None, None),
                   P(None, None)),
        check_vma=False,
    )(x3, codebook)

    n_tok = jnp.float32(B * HW)
    loss = (COMMITMENT_COST / (n_tok * EMB_DIM)) * lsum[0, 0]
    avg = counts / n_tok
    ent = jnp.sum(avg * jnp.log(avg + 1e-10)) / 8.0
    perplexity = jnp.exp(-ent)
    quantized_st = q3.reshape(B, C, H, W)
    return loss, quantized_st, perplexity, oh


# speedup vs baseline: 7.6368x; 7.6368x over previous
import jax
import jax.numpy as jnp
from jax.experimental import pallas as pl
from jax.experimental.pallas import tpu as pltpu

N_EMB = 1024
EMB_DIM = 64
COMMITMENT_COST = 0.25


def _vq_body(x_ref, cb_ref, oh_ref, q_ref, loss_ref, counts_ref):
    b = pl.program_id(0)
    xT = x_ref[0]
    cb = cb_ref[...]

    sx = jnp.sum(xT * xT, axis=0)
    scb = jnp.sum(cb * cb, axis=1)
    m = jax.lax.dot_general(cb, xT, (((1,), (0,)), ((), ())),
                            preferred_element_type=jnp.float32)
    dist = (sx[None, :] + scb[:, None]) - 2.0 * m
    idx = jnp.argmin(dist, axis=0)

    eiota = jax.lax.broadcasted_iota(jnp.int32, dist.shape, 0)
    ohT = (eiota == idx[None, :]).astype(jnp.float32)
    oh_ref[0] = ohT

    qT = jax.lax.dot_general(cb, ohT, (((0,), (0,)), ((), ())),
                             preferred_element_type=jnp.float32)
    q_ref[0] = qT

    part_loss = jnp.sum((qT - xT) ** 2).reshape(1, 1)
    ones_n = jnp.ones((ohT.shape[1], 8), jnp.float32)
    part_counts = jax.lax.dot_general(ohT, ones_n, (((1,), (0,)), ((), ())),
                                      preferred_element_type=jnp.float32)

    @pl.when(b == 0)
    def _init():
        loss_ref[...] = part_loss
        counts_ref[...] = part_counts

    @pl.when(b > 0)
    def _acc():
        loss_ref[...] += part_loss
        counts_ref[...] += part_counts


def kernel(inputs, codebook):
    B, C, H, W = inputs.shape
    HW = H * W
    x3 = inputs.reshape(B, C, HW)

    oh, q3, lsum, counts = pl.pallas_call(
        _vq_body,
        grid=(B,),
        in_specs=[
            pl.BlockSpec((1, C, HW), lambda b: (b, 0, 0)),
            pl.BlockSpec((N_EMB, EMB_DIM), lambda b: (0, 0)),
        ],
        out_specs=[
            pl.BlockSpec((1, N_EMB, HW), lambda b: (b, 0, 0)),
            pl.BlockSpec((1, C, HW), lambda b: (b, 0, 0)),
            pl.BlockSpec((1, 1), lambda b: (0, 0)),
            pl.BlockSpec((N_EMB, 8), lambda b: (0, 0)),
        ],
        out_shape=[
            jax.ShapeDtypeStruct((B, N_EMB, HW), jnp.float32),
            jax.ShapeDtypeStruct((B, C, HW), jnp.float32),
            jax.ShapeDtypeStruct((1, 1), jnp.float32),
            jax.ShapeDtypeStruct((N_EMB, 8), jnp.float32),
        ],
        compiler_params=pltpu.CompilerParams(
            dimension_semantics=("arbitrary",),
        ),
    )(x3, codebook)

    n_tok = jnp.float32(B * HW)
    loss = (COMMITMENT_COST / (n_tok * EMB_DIM)) * lsum[0, 0]
    avg = counts / n_tok
    ent = jnp.sum(avg * jnp.log(avg + 1e-10)) / 8.0
    perplexity = jnp.exp(-ent)
    quantized_st = q3.reshape(B, C, H, W)
    return loss, quantized_st, perplexity, oh
